# baseline (speedup 1.0000x reference)
import functools

import jax
import jax.numpy as jnp
from jax import lax
from jax.experimental import pallas as pl
from jax.experimental.pallas import tpu as pltpu
from jax.experimental.pallas import tpu_sc as plsc

_B = 16384
_D = 64
_W = 2 * _D
_N = 1000000
_NP = 512 * ((_N + 1023) // 1024)

_NC, _NS = 2, 16
_NW = _NC * _NS
_BPW = _B // _NW
_CHUNK = 128
_NCHUNK = _BPW // _CHUNK

_LBLK = 1024


def _pack_body(tabT_ref, out_ref):
    blk = tabT_ref[...]
    lo = blk[:, :_LBLK // 2].T
    hi = blk[:, _LBLK // 2:].T
    out_ref[...] = jnp.concatenate((lo, hi), axis=1)


def _pack_call(tabT):
    import math
    grid = math.ceil(_N / _LBLK)
    return pl.pallas_call(
        _pack_body,
        grid=(grid,),
        in_specs=[pl.BlockSpec((_D, _LBLK), lambda i: (0, i))],
        out_specs=pl.BlockSpec((_LBLK // 2, _W), lambda i: (i, 0)),
        out_shape=jax.ShapeDtypeStruct((_NP, _W), jnp.float32),
    )(tabT)


@functools.cache
def _sc_gather_fn():
    @functools.partial(
        pl.kernel,
        mesh=plsc.VectorSubcoreMesh(core_axis_name="c", subcore_axis_name="s",
                                    num_cores=_NC, num_subcores=_NS),
        out_type=jax.ShapeDtypeStruct((_B, _W), jnp.float32),
        scratch_types=[
            pltpu.VMEM((_NCHUNK, _CHUNK), jnp.int32),
            pltpu.VMEM((_BPW, _W), jnp.float32),
            pltpu.SemaphoreType.DMA,
        ],
    )
    def _sc_gather(idx_hbm, tab_hbm, out_hbm, idx_v, rows_v, sem):
        wid = lax.axis_index("s") * _NC + lax.axis_index("c")
        base = wid * _BPW
        pltpu.sync_copy(idx_hbm.at[wid], idx_v)
        copies = [
            pltpu.async_copy(tab_hbm.at[idx_v.at[j]],
                             rows_v.at[pl.ds(j * _CHUNK, _CHUNK)], sem)
            for j in range(_NCHUNK)
        ]
        for cp in copies:
            cp.wait()
        pltpu.sync_copy(rows_v, out_hbm.at[pl.ds(base, _BPW)])

    return _sc_gather


def _mlp_body(gu_ref, gi_ref, up_ref, ip_ref, W1_ref, b1_ref, W2_ref, b2_ref,
              W3_ref, b3_ref, wl_ref, bl_ref, out_ref):
    gu = gu_ref[...]
    gi = gi_ref[...]
    p = jnp.where(up_ref[...] == 1, gu[:, _D:], gu[:, :_D])
    q = jnp.where(ip_ref[...] == 1, gi[:, _D:], gi[:, :_D])
    x = jnp.concatenate((p, q), axis=-1)
    h = jnp.dot(x, W1_ref[...], preferred_element_type=jnp.float32) + b1_ref[...]
    h = jnp.where(h > 0, h, 0.01 * h)
    h = jnp.dot(h, W2_ref[...], preferred_element_type=jnp.float32) + b2_ref[...]
    h = jnp.where(h > 0, h, 0.01 * h)
    m = jnp.dot(h, W3_ref[...], preferred_element_type=jnp.float32) + b3_ref[...]
    mf = jnp.concatenate((p * q, m), axis=-1)
    out_ref[...] = (jnp.sum(mf * wl_ref[...], axis=-1, keepdims=True)
                    + bl_ref[...])


_BLK = 2048


def _mlp_call(gu, gi, up, ip, W1, b1, W2, b2, W3, b3, wlT, bl):
    full = lambda shape: pl.BlockSpec(shape, lambda i: (0,) * len(shape))
    return pl.pallas_call(
        _mlp_body,
        grid=(_B // _BLK,),
        in_specs=[
            pl.BlockSpec((_BLK, _W), lambda i: (i, 0)),
            pl.BlockSpec((_BLK, _W), lambda i: (i, 0)),
            pl.BlockSpec((_BLK, 1), lambda i: (i, 0)),
            pl.BlockSpec((_BLK, 1), lambda i: (i, 0)),
            full((2 * _D, 256)),
            full((1, 256)),
            full((256, 256)),
            full((1, 256)),
            full((256, _D)),
            full((1, _D)),
            full((1, 2 * _D)),
            full((1, 1)),
        ],
        out_specs=pl.BlockSpec((_BLK, 1), lambda i: (i, 0)),
        out_shape=jax.ShapeDtypeStruct((_B, 1), jnp.float32),
    )(gu, gi, up, ip, W1, b1, W2, b2, W3, b3, wlT, bl)


def kernel(user, item, user_table, item_table, W1, b1, W2, b2, W3, b3, Wl, bl):
    pack_u = _pack_call(user_table.T)
    pack_i = _pack_call(item_table.T)
    u_pr = ((user >> 10) << 9) + (user & 511)
    i_pr = ((item >> 10) << 9) + (item & 511)
    u_r = u_pr.reshape(_NW, _NCHUNK, _CHUNK)
    i_r = i_pr.reshape(_NW, _NCHUNK, _CHUNK)
    gather = _sc_gather_fn()
    gu = gather(u_r, pack_u)
    gi = gather(i_r, pack_i)
    up = ((user >> 9) & 1).reshape(_B, 1)
    ip = ((item >> 9) & 1).reshape(_B, 1)
    return _mlp_call(gu, gi, up, ip, W1, b1.reshape(1, -1), W2,
                     b2.reshape(1, -1), W3, b3.reshape(1, -1),
                     Wl.reshape(1, 2 * _D), bl.reshape(1, 1))
